# baseline (speedup 1.0000x reference)
import functools
import jax
import jax.numpy as jnp
from jax import lax
from jax.experimental import pallas as pl
from jax.experimental.pallas import tpu as pltpu
from jax.experimental.pallas import tpu_sc as plsc

_D = 64
_W = 256
_NW = 32
_N = 1_000_000
_FULL = _N // _W
_TAIL = _N - _FULL * _W
_JMAX = -(-_FULL // _NW)


def _rsqrt16(s):
    i = lax.bitcast_convert_type(s, jnp.int32)
    i = jnp.int32(0x5F3759DF) - lax.shift_right_logical(i, 1)
    y = lax.bitcast_convert_type(i, jnp.float32)
    for _ in range(3):
        y = y * (1.5 - 0.5 * s * y * y)
    return y


def _norm_cols(x_v, o_v, w):
    for g in range(w // 16):
        sl = pl.ds(g * 16, 16)
        t = x_v[0, sl]
        acc = t * t
        for f in range(1, _D):
            t = x_v[f, sl]
            acc = acc + t * t
        inv = _rsqrt16(jnp.maximum(acc, 1e-24))
        for f in range(_D):
            o_v[f, sl] = x_v[f, sl] * inv


def _sc_body(x_hbm, o_hbm, x_v, o_v, x_t, o_t):
    wid = lax.axis_index("s") * 2 + lax.axis_index("c")

    @pl.loop(0, _JMAX)
    def _(j):
        k = j * _NW + wid

        @pl.when(k < _FULL)
        def _():
            c0 = k * _W
            pltpu.sync_copy(x_hbm.at[:, pl.ds(c0, _W)], x_v)
            _norm_cols(x_v, o_v, _W)
            pltpu.sync_copy(o_v, o_hbm.at[:, pl.ds(c0, _W)])

    @pl.when(wid == 1)
    def _():
        c0 = _FULL * _W
        pltpu.sync_copy(x_hbm.at[:, pl.ds(c0, _TAIL)], x_t)
        _norm_cols(x_t, o_t, _TAIL)
        pltpu.sync_copy(o_t, o_hbm.at[:, pl.ds(c0, _TAIL)])


def kernel(prototypes):
    n, d = prototypes.shape
    xt = prototypes.T
    run = pl.kernel(
        _sc_body,
        out_type=jax.ShapeDtypeStruct((d, n), jnp.float32),
        mesh=plsc.VectorSubcoreMesh(core_axis_name="c", subcore_axis_name="s"),
        scratch_types=[
            pltpu.VMEM((_D, _W), jnp.float32),
            pltpu.VMEM((_D, _W), jnp.float32),
            pltpu.VMEM((_D, _TAIL), jnp.float32),
            pltpu.VMEM((_D, _TAIL), jnp.float32),
        ],
    )
    return run(xt).T


# speedup vs baseline: 4.3379x; 4.3379x over previous
import jax
import jax.numpy as jnp
from jax.experimental import pallas as pl

_BLOCK_COLS = 49152


def _normalize_block(x_ref, o_ref):
    x = x_ref[...]
    s = jnp.sum(x * x, axis=0, keepdims=True)
    o_ref[...] = x * jax.lax.rsqrt(jnp.maximum(s, 1e-24))


def kernel(prototypes):
    n, d = prototypes.shape
    xt = prototypes.T
    out_t = pl.pallas_call(
        _normalize_block,
        grid=(pl.cdiv(n, _BLOCK_COLS),),
        in_specs=[pl.BlockSpec((d, _BLOCK_COLS), lambda i: (0, i))],
        out_specs=pl.BlockSpec((d, _BLOCK_COLS), lambda i: (0, i)),
        out_shape=jax.ShapeDtypeStruct((d, n), prototypes.dtype),
    )(xt)
    return out_t.T
